# baseline (speedup 1.0000x reference)
import functools

import jax
import jax.numpy as jnp
from jax import lax
from jax.experimental import pallas as pl
from jax.experimental.pallas import tpu as pltpu
from jax.experimental.pallas import tpu_sc as plsc

N = 10000
E = 320000
HALF = 5000
SROW = 15360
DEGW = 30720

EPT = E // 16
IC = 800
NCH = EPT // IC
KB = 32
NBLK = IC // KB
ZROWS = DEGW // 16

_MESH = plsc.VectorSubcoreMesh(core_axis_name="c", subcore_axis_name="s")


def _prep_body(d_ref, ei_ref, ga1_ref, ga2_ref, sidx_ref):
    d = d_ref[...]
    mn = jnp.min(d)
    mx = jnp.max(d)
    bs = (mx - mn) / 3.0
    t = jnp.clip(jnp.floor((d - mn) / bs), 0.0, 2.0).astype(jnp.int32)
    src = ei_ref[0]
    dst = ei_ref[1]
    ga1_ref[0] = 2 * src
    ga1_ref[1] = 2 * src + 1
    g2 = t * N + src
    ga2_ref[0] = 2 * g2
    ga2_ref[1] = 2 * g2 + 1
    h = (dst >= HALF).astype(jnp.int32)
    sidx_ref[...] = h * SROW + t * 5120 + (dst - h * HALF)


def _prep(d2, ei3):
    return pl.pallas_call(
        _prep_body,
        out_shape=[
            jax.ShapeDtypeStruct((2, 2500, 128), jnp.int32),
            jax.ShapeDtypeStruct((2, 2500, 128), jnp.int32),
            jax.ShapeDtypeStruct((2500, 128), jnp.int32),
        ],
    )(d2, ei3)


@functools.partial(
    pl.kernel,
    out_type=jax.ShapeDtypeStruct((32, DEGW), jnp.float32),
    mesh=_MESH,
    compiler_params=pltpu.CompilerParams(needs_layout_passes=False),
    scratch_types=[
        pltpu.VMEM((DEGW,), jnp.float32),
        pltpu.VMEM((2000,), jnp.int32),
    ],
)
def _deg_kernel(sidx_hbm, out_hbm, accv, sidxv):
    c = lax.axis_index("c")
    s = lax.axis_index("s")
    wid = s * 2 + c
    ones = jnp.ones((16,), jnp.float32)
    zeros = jnp.zeros((16,), jnp.float32)

    def zero_body(i, _):
        accv[pl.ds(i * 16, 16)] = zeros
        return 0

    lax.fori_loop(0, DEGW // 16, zero_body, 0)

    base = wid * (E // 32)

    def blk(b, _):
        pltpu.sync_copy(sidx_hbm.at[pl.ds(base + b * 2000, 2000)], sidxv)

        def grp(g, _):
            iv = sidxv[pl.ds(g * 16, 16)]
            plsc.addupdate_scatter(accv, [iv], ones)
            return 0

        lax.fori_loop(0, 125, grp, 0)
        return 0

    lax.fori_loop(0, 5, blk, 0)
    pltpu.sync_copy(accv, out_hbm.at[wid])


@functools.partial(
    pl.kernel,
    out_type=jax.ShapeDtypeStruct((DEGW, 2, 64), jnp.float32),
    mesh=_MESH,
    compiler_params=pltpu.CompilerParams(use_tc_tiling_on_sc=False),
    scratch_types=[
        pltpu.VMEM_SHARED((DEGW, 64), jnp.float32),
        pltpu.VMEM((3 * KB, 64), jnp.float32),
        pltpu.VMEM((IC,), jnp.int32),
        pltpu.VMEM((NBLK, KB), jnp.int32),
        pltpu.SemaphoreType.DMA,
        pltpu.SemaphoreType.DMA,
    ],
)
def _scatter_kernel(table_hbm, ga_hbm, sidx_hbm, out_hbm, acc, rows, gv, sv,
                    semg, sems):
    c = lax.axis_index("c")
    s = lax.axis_index("s")
    zeros = jnp.zeros((16,), jnp.float32)

    def zr(i, _):
        for j in range(4):
            rows[i, pl.ds(j * 16, 16)] = zeros
        return 0

    lax.fori_loop(0, 3 * KB, zr, 0)

    zb = s * ZROWS
    zd = [pltpu.async_copy(rows, acc.at[pl.ds(zb + k * 3 * KB, 3 * KB)], semg)
          for k in range(ZROWS // (3 * KB))]
    for d in zd:
        d.wait()
    plsc.subcore_barrier()

    base = s * EPT

    def chunk(ch, _):
        off = base + ch * IC
        pltpu.sync_copy(ga_hbm.at[pl.ds(c * E + off, IC)], gv)
        pltpu.sync_copy(sidx_hbm.at[pl.ds(off // KB, NBLK), :], sv)

        def gather(b):
            p = b % 3
            return pltpu.async_copy(
                table_hbm.at[gv.at[pl.ds(b * KB, KB)]],
                rows.at[pl.ds(p * KB, KB)], semg)

        def scatter(b):
            p = b % 3
            return pltpu.async_copy(
                rows.at[pl.ds(p * KB, KB)],
                acc.at[sv.at[b]], sems, add=True)

        gd = {0: gather(0), 1: gather(1)}
        sd = {}
        for b in range(NBLK):
            gd.pop(b).wait()
            sd[b] = scatter(b)
            if b + 2 < NBLK:
                if b >= 1:
                    sd.pop(b - 1).wait()
                gd[b + 2] = gather(b + 2)
        for b in sorted(sd):
            sd.pop(b).wait()
        return 0

    lax.fori_loop(0, NCH, chunk, 0)
    plsc.subcore_barrier()
    pltpu.sync_copy(acc.at[pl.ds(s * ZROWS, ZROWS)],
                    out_hbm.at[pl.ds(s * ZROWS, ZROWS), c, :])


def _rgcn1_body(x_ref, a1_ref, deg_ref, w1_ref, w1s_ref, b1_ref, h1_ref):
    for h in range(2):
        rows = slice(h * HALF, (h + 1) * HALF)
        xh = x_ref[rows, :]
        h1_ref[rows, :] = (
            jnp.dot(xh, w1s_ref[...], preferred_element_type=jnp.float32)
            + b1_ref[...])
        for t in range(3):
            iv = (1.0 / jnp.maximum(deg_ref[h, t, :HALF], 1.0))[:, None]
            sl = slice(h * SROW + t * 5120, h * SROW + t * 5120 + HALF)
            h1_ref[rows, :] += jnp.dot(a1_ref[sl, :] * iv, w1_ref[t],
                                       preferred_element_type=jnp.float32)
        h1_ref[rows, :] = jnp.maximum(h1_ref[rows, :], 0.0)


def _rgcn1(x, A1, deg3, W1, W1_self, b1):
    return pl.pallas_call(
        _rgcn1_body,
        out_shape=jax.ShapeDtypeStruct((N, 256), jnp.float32),
    )(x, A1, deg3, W1, W1_self, b1)


def _degsum_body(degp_ref, deg_ref):
    deg_ref[...] = jnp.sum(degp_ref[...], axis=0)


def _degsum(degp4):
    return pl.pallas_call(
        _degsum_body,
        out_shape=jax.ShapeDtypeStruct((2, 3, 5120), jnp.float32),
    )(degp4)


def _proj2_body(h1_ref, w2_ref, w2s_ref, b2_ref, s2_ref, y_ref):
    h1 = h1_ref[...]
    s2_ref[...] = jnp.dot(h1, w2s_ref[...], preferred_element_type=jnp.float32) + b2_ref[...]
    for t in range(3):
        y_ref[t] = jnp.dot(h1, w2_ref[t], preferred_element_type=jnp.float32)


def _proj2(h1, W2, W2_self, b2):
    return pl.pallas_call(
        _proj2_body,
        grid=(2,),
        in_specs=[
            pl.BlockSpec((HALF, 256), lambda h: (h, 0)),
            pl.BlockSpec((3, 256, 128), lambda h: (0, 0, 0)),
            pl.BlockSpec((256, 128), lambda h: (0, 0)),
            pl.BlockSpec((1, 128), lambda h: (0, 0)),
        ],
        out_specs=[
            pl.BlockSpec((HALF, 128), lambda h: (h, 0)),
            pl.BlockSpec((3, HALF, 128), lambda h: (0, h, 0)),
        ],
        out_shape=[
            jax.ShapeDtypeStruct((N, 128), jnp.float32),
            jax.ShapeDtypeStruct((3, N, 128), jnp.float32),
        ],
    )(h1, W2, W2_self, b2)


def _dense2_body(s2_ref, a2_ref, deg_ref, node_ref):
    inv = 1.0 / jnp.maximum(deg_ref[0], 1.0)
    nb = s2_ref[...]
    for t in range(3):
        nb = nb + a2_ref[pl.ds(t * 5120, HALF), :] * inv[t, :HALF][:, None]
    node_ref[...] = nb


def _dense2(S2, A2, deg3):
    return pl.pallas_call(
        _dense2_body,
        grid=(2,),
        in_specs=[
            pl.BlockSpec((HALF, 128), lambda h: (h, 0)),
            pl.BlockSpec((SROW, 128), lambda h: (h, 0)),
            pl.BlockSpec((1, 3, 5120), lambda h: (h, 0, 0)),
        ],
        out_specs=pl.BlockSpec((HALF, 128), lambda h: (h, 0)),
        out_shape=jax.ShapeDtypeStruct((N, 128), jnp.float32),
    )(S2, A2, deg3)


@functools.partial(
    pl.kernel,
    out_type=jax.ShapeDtypeStruct((2048, 128), jnp.float32),
    mesh=_MESH,
    scratch_types=[
        pltpu.VMEM((64,), jnp.int32),
        pltpu.VMEM((64, 128), jnp.float32),
        pltpu.SemaphoreType.DMA,
    ],
)
def _gather_kernel(node_hbm, idx_hbm, out_hbm, idxv, rowsv, sem):
    wid = lax.axis_index("s") * 2 + lax.axis_index("c")
    base = wid * 64
    pltpu.sync_copy(idx_hbm.at[pl.ds(base, 64)], idxv)
    pltpu.async_copy(node_hbm.at[idxv], rowsv, sem).wait()
    pltpu.sync_copy(rowsv, out_hbm.at[pl.ds(base, 64)])


def _final_body(g_ref, fcw_ref, fcb_ref, o_ref):
    gn = g_ref[0:1024, :]
    gf = g_ref[1024:2048, :]
    o = (jnp.dot(gn, fcw_ref[0:128, :], preferred_element_type=jnp.float32)
         + jnp.dot(gf, fcw_ref[128:256, :], preferred_element_type=jnp.float32)
         + fcb_ref[...])
    o_ref[...] = jnp.tanh(o)


def _final(g, fc_W, fc_b):
    return pl.pallas_call(
        _final_body,
        out_shape=jax.ShapeDtypeStruct((1024, 256), jnp.float32),
    )(g, fc_W, fc_b)


def kernel(x, edge_index, edge_attr, nest, food, W1, W1_self, b1, W2, W2_self,
           b2, fc_W, fc_b):
    d2 = edge_attr[:, 0].reshape(2500, 128)
    ei3 = edge_index.astype(jnp.int32).reshape(2, 2500, 128)
    gA1, gA2, sidx_2d = _prep(d2, ei3)
    gA1 = gA1.reshape(2 * E)
    gA2 = gA2.reshape(2 * E)
    sidx = sidx_2d.reshape(E)
    sidx32 = sidx_2d.reshape(E // KB, KB)

    degp = _deg_kernel(sidx)
    deg3 = _degsum(degp.reshape(32, 2, 3, 5120))

    A1 = _scatter_kernel(x.reshape(2 * N, 64), gA1, sidx32).reshape(DEGW, 128)
    h1 = _rgcn1(x, A1, deg3, W1, W1_self, b1.reshape(1, 256))
    S2, Y = _proj2(h1, W2, W2_self, b2.reshape(1, 128))
    A2 = _scatter_kernel(Y.reshape(6 * N, 64), gA2, sidx32).reshape(DEGW, 128)
    node = _dense2(S2, A2, deg3)

    idx = jnp.concatenate([nest.astype(jnp.int32), food.astype(jnp.int32)])
    g = _gather_kernel(node, idx)
    return _final(g, fc_W, fc_b.reshape(1, 256))


# speedup vs baseline: 1.2109x; 1.2109x over previous
import functools

import jax
import jax.numpy as jnp
from jax import lax
from jax.experimental import pallas as pl
from jax.experimental.pallas import tpu as pltpu
from jax.experimental.pallas import tpu_sc as plsc

N = 10000
E = 320000
HALF = 5000
SROW = 15360
DEGW = 30720

EPT = E // 16
IC = 800
NCH = EPT // IC
KB = 32
NBLK = IC // KB
ZROWS = DEGW // 16

_MESH = plsc.VectorSubcoreMesh(core_axis_name="c", subcore_axis_name="s")


def _prep_body(d_ref, ei_ref, ga1_ref, ga2_ref, sidx_ref):
    d = d_ref[...]
    mn = jnp.min(d)
    mx = jnp.max(d)
    bs = (mx - mn) / 3.0
    t = jnp.clip(jnp.floor((d - mn) / bs), 0.0, 2.0).astype(jnp.int32)
    src = ei_ref[0]
    dst = ei_ref[1]
    ga1_ref[0] = 2 * src
    ga1_ref[1] = 2 * src + 1
    g2 = t * N + src
    ga2_ref[0] = 2 * g2
    ga2_ref[1] = 2 * g2 + 1
    h = (dst >= HALF).astype(jnp.int32)
    sidx_ref[...] = h * SROW + t * 5120 + (dst - h * HALF)


def _prep(d2, ei3):
    return pl.pallas_call(
        _prep_body,
        out_shape=[
            jax.ShapeDtypeStruct((2, 2500, 128), jnp.int32),
            jax.ShapeDtypeStruct((2, 2500, 128), jnp.int32),
            jax.ShapeDtypeStruct((2500, 128), jnp.int32),
        ],
    )(d2, ei3)


@functools.partial(
    pl.kernel,
    out_type=jax.ShapeDtypeStruct((32, DEGW), jnp.float32),
    mesh=_MESH,
    compiler_params=pltpu.CompilerParams(needs_layout_passes=False),
    scratch_types=[
        pltpu.VMEM((DEGW,), jnp.float32),
        pltpu.VMEM((2000,), jnp.int32),
    ],
)
def _deg_kernel(sidx_hbm, out_hbm, accv, sidxv):
    c = lax.axis_index("c")
    s = lax.axis_index("s")
    wid = s * 2 + c
    ones = jnp.ones((16,), jnp.float32)
    zeros = jnp.zeros((16,), jnp.float32)

    def zero_body(i, _):
        accv[pl.ds(i * 16, 16)] = zeros
        return 0

    lax.fori_loop(0, DEGW // 16, zero_body, 0)

    base = wid * (E // 32)

    def blk(b, _):
        pltpu.sync_copy(sidx_hbm.at[pl.ds(base + b * 2000, 2000)], sidxv)

        def grp(g, _):
            iv = sidxv[pl.ds(g * 16, 16)]
            plsc.addupdate_scatter(accv, [iv], ones)
            return 0

        lax.fori_loop(0, 125, grp, 0)
        return 0

    lax.fori_loop(0, 5, blk, 0)
    pltpu.sync_copy(accv, out_hbm.at[wid])


@functools.partial(
    pl.kernel,
    out_type=jax.ShapeDtypeStruct((2 * DEGW, 64), jnp.float32),
    mesh=_MESH,
    compiler_params=pltpu.CompilerParams(use_tc_tiling_on_sc=False),
    scratch_types=[
        pltpu.VMEM_SHARED((DEGW, 64), jnp.float32),
        pltpu.VMEM((3 * KB, 64), jnp.float32),
        pltpu.VMEM((IC,), jnp.int32),
        pltpu.VMEM((NBLK, KB), jnp.int32),
        pltpu.SemaphoreType.DMA,
        pltpu.SemaphoreType.DMA,
    ],
)
def _scatter_kernel(table_hbm, ga_hbm, sidx_hbm, out_hbm, acc, rows, gv, sv,
                    semg, sems):
    c = lax.axis_index("c")
    s = lax.axis_index("s")
    zeros = jnp.zeros((16,), jnp.float32)

    def zr(i, _):
        for j in range(4):
            rows[i, pl.ds(j * 16, 16)] = zeros
        return 0

    lax.fori_loop(0, 3 * KB, zr, 0)

    zb = s * ZROWS
    zd = [pltpu.async_copy(rows, acc.at[pl.ds(zb + k * 3 * KB, 3 * KB)], semg)
          for k in range(ZROWS // (3 * KB))]
    for d in zd:
        d.wait()
    plsc.subcore_barrier()

    base = s * EPT

    def chunk(ch, _):
        off = base + ch * IC
        pltpu.sync_copy(ga_hbm.at[pl.ds(c * E + off, IC)], gv)
        pltpu.sync_copy(sidx_hbm.at[pl.ds(off // KB, NBLK), :], sv)

        def gather(b):
            p = b % 3
            return pltpu.async_copy(
                table_hbm.at[gv.at[pl.ds(b * KB, KB)]],
                rows.at[pl.ds(p * KB, KB)], semg)

        def scatter(b):
            p = b % 3
            return pltpu.async_copy(
                rows.at[pl.ds(p * KB, KB)],
                acc.at[sv.at[b]], sems, add=True)

        gd = {0: gather(0), 1: gather(1)}
        sd = {}
        for b in range(NBLK):
            gd.pop(b).wait()
            sd[b] = scatter(b)
            if b + 2 < NBLK:
                if b >= 1:
                    sd.pop(b - 1).wait()
                gd[b + 2] = gather(b + 2)
        for b in sorted(sd):
            sd.pop(b).wait()
        return 0

    lax.fori_loop(0, NCH, chunk, 0)
    plsc.subcore_barrier()
    pltpu.sync_copy(acc.at[pl.ds(s * ZROWS, ZROWS)],
                    out_hbm.at[pl.ds(c * DEGW + s * ZROWS, ZROWS)])


def _rgcn1_body(x_ref, a1_ref, deg_ref, w1_ref, w1s_ref, b1_ref, h1_ref):
    for h in range(2):
        rows = slice(h * HALF, (h + 1) * HALF)
        xh = x_ref[rows, :]
        h1_ref[rows, :] = (
            jnp.dot(xh, w1s_ref[...], preferred_element_type=jnp.float32)
            + b1_ref[...])
        for t in range(3):
            iv = (1.0 / jnp.maximum(deg_ref[h, t, :HALF], 1.0))[:, None]
            sl = slice(h * SROW + t * 5120, h * SROW + t * 5120 + HALF)
            h1_ref[rows, :] += (
                jnp.dot(a1_ref[0, sl, :] * iv, w1_ref[t, 0:64, :],
                        preferred_element_type=jnp.float32)
                + jnp.dot(a1_ref[1, sl, :] * iv, w1_ref[t, 64:128, :],
                          preferred_element_type=jnp.float32))
        h1_ref[rows, :] = jnp.maximum(h1_ref[rows, :], 0.0)


def _rgcn1(x, A1, deg3, W1, W1_self, b1):
    return pl.pallas_call(
        _rgcn1_body,
        out_shape=jax.ShapeDtypeStruct((N, 256), jnp.float32),
    )(x, A1, deg3, W1, W1_self, b1)


def _degsum_body(degp_ref, deg_ref):
    deg_ref[...] = jnp.sum(degp_ref[...], axis=0)


def _degsum(degp4):
    return pl.pallas_call(
        _degsum_body,
        out_shape=jax.ShapeDtypeStruct((2, 3, 5120), jnp.float32),
    )(degp4)


def _proj2_body(h1_ref, w2_ref, w2s_ref, b2_ref, s2_ref, y_ref):
    h1 = h1_ref[...]
    s2_ref[...] = jnp.dot(h1, w2s_ref[...], preferred_element_type=jnp.float32) + b2_ref[...]
    for t in range(3):
        y_ref[t] = jnp.dot(h1, w2_ref[t], preferred_element_type=jnp.float32)


def _proj2(h1, W2, W2_self, b2):
    return pl.pallas_call(
        _proj2_body,
        grid=(2,),
        in_specs=[
            pl.BlockSpec((HALF, 256), lambda h: (h, 0)),
            pl.BlockSpec((3, 256, 128), lambda h: (0, 0, 0)),
            pl.BlockSpec((256, 128), lambda h: (0, 0)),
            pl.BlockSpec((1, 128), lambda h: (0, 0)),
        ],
        out_specs=[
            pl.BlockSpec((HALF, 128), lambda h: (h, 0)),
            pl.BlockSpec((3, HALF, 128), lambda h: (0, h, 0)),
        ],
        out_shape=[
            jax.ShapeDtypeStruct((N, 128), jnp.float32),
            jax.ShapeDtypeStruct((3, N, 128), jnp.float32),
        ],
    )(h1, W2, W2_self, b2)


def _dense2_body(s2_ref, a2_ref, deg_ref, node_ref):
    inv = 1.0 / jnp.maximum(deg_ref[0], 1.0)
    for half in range(2):
        cs = pl.ds(half * 64, 64)
        nb = s2_ref[:, cs]
        for t in range(3):
            nb = nb + a2_ref[half, pl.ds(t * 5120, HALF), :] * inv[t, :HALF][:, None]
        node_ref[:, cs] = nb


def _dense2(S2, A2, deg3):
    return pl.pallas_call(
        _dense2_body,
        grid=(2,),
        in_specs=[
            pl.BlockSpec((HALF, 128), lambda h: (h, 0)),
            pl.BlockSpec((2, SROW, 64), lambda h: (0, h, 0)),
            pl.BlockSpec((1, 3, 5120), lambda h: (h, 0, 0)),
        ],
        out_specs=pl.BlockSpec((HALF, 128), lambda h: (h, 0)),
        out_shape=jax.ShapeDtypeStruct((N, 128), jnp.float32),
    )(S2, A2, deg3)


@functools.partial(
    pl.kernel,
    out_type=jax.ShapeDtypeStruct((2048, 128), jnp.float32),
    mesh=_MESH,
    scratch_types=[
        pltpu.VMEM((64,), jnp.int32),
        pltpu.VMEM((64, 128), jnp.float32),
        pltpu.SemaphoreType.DMA,
    ],
)
def _gather_kernel(node_hbm, idx_hbm, out_hbm, idxv, rowsv, sem):
    wid = lax.axis_index("s") * 2 + lax.axis_index("c")
    base = wid * 64
    pltpu.sync_copy(idx_hbm.at[pl.ds(base, 64)], idxv)
    pltpu.async_copy(node_hbm.at[idxv], rowsv, sem).wait()
    pltpu.sync_copy(rowsv, out_hbm.at[pl.ds(base, 64)])


def _final_body(g_ref, fcw_ref, fcb_ref, o_ref):
    gn = g_ref[0:1024, :]
    gf = g_ref[1024:2048, :]
    o = (jnp.dot(gn, fcw_ref[0:128, :], preferred_element_type=jnp.float32)
         + jnp.dot(gf, fcw_ref[128:256, :], preferred_element_type=jnp.float32)
         + fcb_ref[...])
    o_ref[...] = jnp.tanh(o)


def _final(g, fc_W, fc_b):
    return pl.pallas_call(
        _final_body,
        out_shape=jax.ShapeDtypeStruct((1024, 256), jnp.float32),
    )(g, fc_W, fc_b)


def kernel(x, edge_index, edge_attr, nest, food, W1, W1_self, b1, W2, W2_self,
           b2, fc_W, fc_b):
    d2 = edge_attr[:, 0].reshape(2500, 128)
    ei3 = edge_index.astype(jnp.int32).reshape(2, 2500, 128)
    gA1, gA2, sidx_2d = _prep(d2, ei3)
    gA1 = gA1.reshape(2 * E)
    gA2 = gA2.reshape(2 * E)
    sidx = sidx_2d.reshape(E)
    sidx32 = sidx_2d.reshape(E // KB, KB)

    degp = _deg_kernel(sidx)
    deg3 = _degsum(degp.reshape(32, 2, 3, 5120))

    A1 = _scatter_kernel(x.reshape(2 * N, 64), gA1, sidx32).reshape(2, DEGW, 64)
    h1 = _rgcn1(x, A1, deg3, W1, W1_self, b1.reshape(1, 256))
    S2, Y = _proj2(h1, W2, W2_self, b2.reshape(1, 128))
    A2 = _scatter_kernel(Y.reshape(6 * N, 64), gA2, sidx32).reshape(2, DEGW, 64)
    node = _dense2(S2, A2, deg3)

    idx = jnp.concatenate([nest.astype(jnp.int32), food.astype(jnp.int32)])
    g = _gather_kernel(node, idx)
    return _final(g, fc_W, fc_b.reshape(1, 256))
